# baseline (speedup 1.0000x reference)
import functools

import jax
import jax.numpy as jnp
from jax import lax
from jax.experimental import pallas as pl
from jax.experimental.pallas import tpu as pltpu
from jax.experimental.pallas import tpu_sc as plsc

_B = 16384
_M = 21
_D = 64
_NC = 2
_NS = 16
_NW = _NC * _NS
_BW = _B // _NW
_CQ = 32
_NCHUNK = _BW // _CQ
_P = _CQ * _M
_IW = 112
_IR = _P // _IW


def _body(qidx_hbm, midx_hbm, itab_hbm, otab_hbm, out_hbm,
          qidx_v, midx_v, hid_v, mat_v, logit_v, stage_v, sem):
    cid = lax.axis_index("c")
    sid = lax.axis_index("s")
    wid = sid * _NC + cid
    lane = lax.iota(jnp.int32, 16)

    pltpu.sync_copy(qidx_hbm.at[pl.ds(wid * _NCHUNK, _NCHUNK)], qidx_v)
    pltpu.sync_copy(midx_hbm.at[pl.ds(wid * _NCHUNK * _IR, _NCHUNK * _IR)],
                    midx_v)

    @pl.loop(0, _NCHUNK)
    def _chunk(c):
        pltpu.async_copy(itab_hbm.at[qidx_v.at[c]], hid_v, sem).wait()
        for j in range(_IR):
            pltpu.async_copy(otab_hbm.at[midx_v.at[c * _IR + j]],
                             mat_v.at[pl.ds(j * _IW, _IW)], sem).wait()

        @pl.loop(0, _CQ)
        def _qloop(q):
            h0 = hid_v[q, pl.ds(0, 16)]
            h1 = hid_v[q, pl.ds(16, 16)]
            h2 = hid_v[q, pl.ds(32, 16)]
            h3 = hid_v[q, pl.ds(48, 16)]
            base = q * _M
            for g, cnt in ((0, 16), (16, _M - 16)):
                for i in range(cnt):
                    p = base + g + i
                    a = (mat_v[p, pl.ds(0, 16)] * h0
                         + mat_v[p, pl.ds(16, 16)] * h1
                         + mat_v[p, pl.ds(32, 16)] * h2
                         + mat_v[p, pl.ds(48, 16)] * h3)
                    stage_v[i, pl.ds(0, 16)] = a
                s = plsc.load_gather(stage_v, [lane, jnp.zeros((16,), jnp.int32)])
                for col in range(1, 16):
                    s = s + plsc.load_gather(
                        stage_v, [lane, jnp.full((16,), col, jnp.int32)])
                plsc.store_scatter(
                    logit_v, [jnp.full((16,), base + g, jnp.int32) + lane], s,
                    mask=(lane < cnt))

        pltpu.sync_copy(
            logit_v, out_hbm.at[pl.ds(wid * _BW * _M + c * _P, _P)])


@jax.jit
def _run(qflat, midx2d, itab, otab):
    mesh = plsc.VectorSubcoreMesh(
        core_axis_name="c", subcore_axis_name="s",
        num_cores=_NC, num_subcores=_NS)
    call = pl.kernel(
        _body,
        out_type=jax.ShapeDtypeStruct((_B * _M,), jnp.float32),
        mesh=mesh,
        compiler_params=pltpu.CompilerParams(
            needs_layout_passes=False, use_tc_tiling_on_sc=False),
        scratch_types=[
            pltpu.VMEM((_NCHUNK, _CQ), jnp.int32),
            pltpu.VMEM((_NCHUNK * _IR, _IW), jnp.int32),
            pltpu.VMEM((_CQ, _D), jnp.float32),
            pltpu.VMEM((_P, _D), jnp.float32),
            pltpu.VMEM((_P,), jnp.float32),
            pltpu.VMEM((16, 17), jnp.float32),
            pltpu.SemaphoreType.DMA,
        ],
    )
    return call(qflat, midx2d, itab, otab)


def kernel(query, match, input_table, output_table):
    qflat = query.astype(jnp.int32).reshape(_B // _CQ, _CQ)
    midx2d = match.astype(jnp.int32).reshape(_B * _M // _IW, _IW)
    out = _run(qflat, midx2d, input_table, output_table)
    return out.reshape(_B, _M)
